# baseline (speedup 1.0000x reference)
import functools

import jax
import jax.numpy as jnp
from jax import lax
from jax.experimental import pallas as pl
from jax.experimental.pallas import tpu as pltpu
from jax.experimental.pallas import tpu_sc as plsc

N_NODES = 50000
N_EDGES = 800000
EMB = 64
NLAYERS = 4
NB = 128

NC, NS = 2, 16
NPAD = 57344
NROWS = NPAD // 128
EPAD = 802816
EROWS = EPAD // 128
ER_PER_TILE = EROWS // NS
ECH = ER_PER_TILE // 4
HALF = N_NODES // 2
AGG_ROWS = 25088
SLAB = AGG_ROWS // NS
TRASH = HALF

_mesh = plsc.VectorSubcoreMesh(
    core_axis_name="c", subcore_axis_name="s", num_cores=NC, num_subcores=NS)
_SC_PARAMS = pltpu.CompilerParams(needs_layout_passes=False,
                                  use_tc_tiling_on_sc=False)


def _atom_body(xt_hbm, atab_hbm, h_hbm, atv, xv0, xv1, hb0, hb1,
               xsem0, xsem1, hsem0, hsem1):
    c = lax.axis_index("c")
    s = lax.axis_index("s")
    wid = s * NC + c
    pltpu.sync_copy(atab_hbm, atv)
    iota = lax.broadcasted_iota(jnp.int32, (16,), 0)
    xv = (xv0, xv1)
    hb = (hb0, hb1)
    xsem = (xsem0, xsem1)
    r0 = wid * 14

    def _compute(xvq, hbq):
        for g in range(8):
            xf = [xvq[f, pl.ds(g * 16, 16)] for f in range(9)]
            xrow = [x >> 1 for x in xf]
            xcol = [(x & 1) << 6 for x in xf]
            rows = g * 16 + iota

            @plsc.parallel_loop(0, EMB, unroll=4)
            def _col(col):
                colv = jnp.full((16,), col, jnp.int32)
                acc = plsc.load_gather(atv, [xrow[0], xcol[0] + colv])
                for f in range(1, 9):
                    acc = acc + plsc.load_gather(atv, [xrow[f], xcol[f] + colv])
                plsc.store_scatter(hbq, [rows, colv], acc)

    @pl.loop(0, 14, step=2)
    def _pair(g):
        dxa = pltpu.async_copy(xt_hbm.at[r0 + g], xv0, xsem0)
        dxb = pltpu.async_copy(xt_hbm.at[r0 + g + 1], xv1, xsem1)
        dxa.wait()
        _compute(xv0, hb0)
        dha = pltpu.async_copy(hb0, h_hbm.at[pl.ds((r0 + g) * 128, 128)],
                               hsem0)
        dxb.wait()
        _compute(xv1, hb1)
        dhb = pltpu.async_copy(hb1, h_hbm.at[pl.ds((r0 + g + 1) * 128, 128)],
                               hsem1)
        dha.wait()
        dhb.wait()


_atom_call = functools.partial(
    pl.kernel,
    out_type=jax.ShapeDtypeStruct((NPAD, EMB), jnp.float32),
    mesh=_mesh,
    compiler_params=_SC_PARAMS,
    scratch_types=[
        pltpu.VMEM((540, 128), jnp.float32),
        pltpu.VMEM((9, 128), jnp.int32),
        pltpu.VMEM((9, 128), jnp.int32),
        pltpu.VMEM((128, EMB), jnp.float32),
        pltpu.VMEM((128, EMB), jnp.float32),
        pltpu.SemaphoreType.DMA,
        pltpu.SemaphoreType.DMA,
        pltpu.SemaphoreType.DMA,
        pltpu.SemaphoreType.DMA,
    ],
)(_atom_body)


NSC = 3136
SC_PER_TILE = NSC // NS


def _edge_body(h_hbm, pk_hbm, ctab_hbm, agg_hbm,
               ctv, idxv0, idxv1, grows0, grows1, zb, aggs,
               gsem0, gsem1, ssem0, ssem1):
    c = lax.axis_index("c")
    s = lax.axis_index("s")
    base = c * HALF
    pltpu.sync_copy(ctab_hbm, ctv)
    zvec = jnp.zeros((16,), jnp.float32)

    @pl.loop(0, 112)
    def _zg(i):
        for q in range(4):
            zb[i, pl.ds(q * 16, 16)] = zvec

    for j in range(14):
        pltpu.sync_copy(zb, aggs.at[pl.ds(s * SLAB + j * 112, 112)])
    plsc.subcore_barrier()

    iota = lax.broadcasted_iota(jnp.int32, (16,), 0)
    idxv = (idxv0, idxv1)
    grows = (grows0, grows1)
    gsem = (gsem0, gsem1)
    ssem = (ssem0, ssem1)
    sc0 = s * SC_PER_TILE

    def _compute(G, ip, b):
        for g4 in range(4):
            sl = pl.ds(g4 * 16, 16)
            d = ip[3 * b + 1, sl]
            own = (d >= base) & (d < base + HALF)
            ip[3 * b + 1, sl] = jnp.where(own, d - base, TRASH + (d & 63))
            cidx16 = ip[3 * b + 2, sl]
            crow = cidx16 >> 1
            ccol = (cidx16 & 1) << 6
            rows = g4 * 16 + iota

            @plsc.parallel_loop(0, EMB, unroll=4)
            def _col(col):
                colv = jnp.full((16,), col, jnp.int32)
                hv = plsc.load_gather(G, [rows, colv])
                ev = plsc.load_gather(ctv, [crow, ccol + colv])
                plsc.store_scatter(G, [rows, colv],
                                   jnp.maximum(hv + ev, 0.0))

    @pl.loop(0, SC_PER_TILE, step=2)
    def _super(g):
        sc = sc0 + g
        pltpu.sync_copy(pk_hbm.at[sc], idxv0)
        pltpu.sync_copy(pk_hbm.at[sc + 1], idxv1)
        dg = [None] * 8
        ds_ = [None] * 8
        dg[0] = pltpu.async_copy(h_hbm.at[idxv0.at[0]], grows0, gsem0)
        for t in range(8):
            q = t & 1
            b = t % 4
            ip = idxv[t // 4]
            if t < 7:
                if t >= 1:
                    ds_[t - 1].wait()
                nip = idxv[(t + 1) // 4]
                nb = (t + 1) % 4
                dg[t + 1] = pltpu.async_copy(h_hbm.at[nip.at[3 * nb]],
                                             grows[1 - q], gsem[1 - q])
            dg[t].wait()
            _compute(grows[q], ip, b)
            ds_[t] = pltpu.async_copy(grows[q], aggs.at[ip.at[3 * b + 1]],
                                      ssem[q], add=True)
        ds_[6].wait()
        ds_[7].wait()

    plsc.subcore_barrier()

    @pl.when(s < NS - 1)
    def _full_slab():
        pltpu.sync_copy(aggs.at[pl.ds(s * SLAB, SLAB)],
                        agg_hbm.at[pl.ds(base + s * SLAB, SLAB)])

    @pl.when(s == NS - 1)
    def _last_slab():
        pltpu.sync_copy(aggs.at[pl.ds((NS - 1) * SLAB, HALF - (NS - 1) * SLAB)],
                        agg_hbm.at[pl.ds(base + (NS - 1) * SLAB,
                                         HALF - (NS - 1) * SLAB)])


_edge_call = functools.partial(
    pl.kernel,
    out_type=jax.ShapeDtypeStruct((NPAD, EMB), jnp.float32),
    mesh=_mesh,
    compiler_params=_SC_PARAMS,
    scratch_types=[
        pltpu.VMEM((64, 128), jnp.float32),
        pltpu.VMEM((12, 64), jnp.int32),
        pltpu.VMEM((12, 64), jnp.int32),
        pltpu.VMEM((64, EMB), jnp.float32),
        pltpu.VMEM((64, EMB), jnp.float32),
        pltpu.VMEM((112, EMB), jnp.float32),
        pltpu.VMEM_SHARED((AGG_ROWS, EMB), jnp.float32),
        pltpu.SemaphoreType.DMA,
        pltpu.SemaphoreType.DMA,
        pltpu.SemaphoreType.DMA,
        pltpu.SemaphoreType.DMA,
    ],
)(_edge_body)


def _pool_body(nr_hbm, b_hbm, pool_hbm, cnt_hbm, vbuf, bv, cntv, zb9, pools):
    c = lax.axis_index("c")
    s = lax.axis_index("s")
    wid = s * NC + c
    iota = lax.broadcasted_iota(jnp.int32, (16,), 0)
    ones = jnp.ones((16,), jnp.float32)
    zvec = jnp.zeros((16,), jnp.float32)

    @pl.loop(0, 144)
    def _zc(i):
        cntv[i, pl.ds(0, 16)] = zvec

    @pl.loop(0, 9)
    def _z9(i):
        for q in range(4):
            zb9[i, pl.ds(q * 16, 16)] = zvec

    pltpu.sync_copy(zb9, pools.at[pl.ds(s * 9, 9)])
    plsc.subcore_barrier()

    @pl.loop(0, 14)
    def _chunk(i):
        r = wid * 14 + i
        pltpu.sync_copy(nr_hbm.at[pl.ds(r * 128, 128)], vbuf)
        pltpu.sync_copy(b_hbm.at[r], bv)
        pltpu.sync_copy(vbuf, pools.at[bv], add=True)
        for g in range(8):
            b16 = bv[pl.ds(g * 16, 16)]
            plsc.addupdate_scatter(cntv, [b16, iota], ones)

    plsc.subcore_barrier()

    @pl.when(s == 0)
    def _out_pool():
        pltpu.sync_copy(pools, pool_hbm.at[c])

    pltpu.sync_copy(cntv, cnt_hbm.at[wid])


_pool_call = functools.partial(
    pl.kernel,
    out_type=(jax.ShapeDtypeStruct((NC, 144, EMB), jnp.float32),
              jax.ShapeDtypeStruct((NC * NS, 144, 16), jnp.float32)),
    mesh=_mesh,
    compiler_params=_SC_PARAMS,
    scratch_types=[
        pltpu.VMEM((128, EMB), jnp.float32),
        pltpu.VMEM((128,), jnp.int32),
        pltpu.VMEM((144, 16), jnp.float32),
        pltpu.VMEM((9, EMB), jnp.float32),
        pltpu.VMEM_SHARED((144, EMB), jnp.float32),
    ],
)(_pool_body)


_BLK = 1024
_GRID = NPAD // _BLK


def _ka_body(eps_ref, h_ref, agg_ref, w_ref, b_ref, z1_ref, s_ref, q_ref):
    i = pl.program_id(0)
    e = eps_ref[0]
    z = (1.0 + e) * h_ref[...] + agg_ref[...]
    z1 = lax.dot_general(z, w_ref[...], (((1,), (0,)), ((), ())),
                         preferred_element_type=jnp.float32) + b_ref[...]
    rows = lax.broadcasted_iota(jnp.int32, (_BLK, 1), 0) + i * _BLK
    z1 = jnp.where(rows < N_NODES, z1, 0.0)
    z1_ref[...] = z1

    @pl.when(i == 0)
    def _():
        s_ref[...] = jnp.zeros_like(s_ref)
        q_ref[...] = jnp.zeros_like(q_ref)

    s_ref[...] += jnp.sum(z1, axis=0, keepdims=True)
    q_ref[...] += jnp.sum(z1 * z1, axis=0, keepdims=True)


def _ka(h, agg, eps, w1, b1):
    e2 = w1.shape[1]
    return pl.pallas_call(
        _ka_body,
        grid=(_GRID,),
        in_specs=[
            pl.BlockSpec(memory_space=pltpu.SMEM),
            pl.BlockSpec((_BLK, EMB), lambda i: (i, 0)),
            pl.BlockSpec((_BLK, EMB), lambda i: (i, 0)),
            pl.BlockSpec((EMB, e2), lambda i: (0, 0)),
            pl.BlockSpec((1, e2), lambda i: (0, 0)),
        ],
        out_specs=[
            pl.BlockSpec((_BLK, e2), lambda i: (i, 0)),
            pl.BlockSpec((1, e2), lambda i: (0, 0)),
            pl.BlockSpec((1, e2), lambda i: (0, 0)),
        ],
        out_shape=[
            jax.ShapeDtypeStruct((NPAD, e2), jnp.float32),
            jax.ShapeDtypeStruct((1, e2), jnp.float32),
            jax.ShapeDtypeStruct((1, e2), jnp.float32),
        ],
    )(eps, h, agg, w1, b1[None])


def _bnfold_body(s_ref, q_ref, g_ref, beta_ref, a_ref, c_ref):
    inv_n = 1.0 / N_NODES
    m = s_ref[...] * inv_n
    v = q_ref[...] * inv_n - m * m
    a = g_ref[...] * lax.rsqrt(v + 1e-5)
    a_ref[...] = a
    c_ref[...] = beta_ref[...] - m * a


def _bnfold(s, q, g, beta):
    e2 = s.shape[1]
    return pl.pallas_call(
        _bnfold_body,
        out_shape=[
            jax.ShapeDtypeStruct((1, e2), jnp.float32),
            jax.ShapeDtypeStruct((1, e2), jnp.float32),
        ],
    )(s, q, g[None], beta[None])


def _kb_body(z1_ref, a_ref, c_ref, w_ref, b_ref, y_ref, s_ref, q_ref):
    i = pl.program_id(0)
    r = jnp.maximum(z1_ref[...] * a_ref[...] + c_ref[...], 0.0)
    rows = lax.broadcasted_iota(jnp.int32, (_BLK, 1), 0) + i * _BLK
    r = jnp.where(rows < N_NODES, r, 0.0)
    y = lax.dot_general(r, w_ref[...], (((1,), (0,)), ((), ())),
                        preferred_element_type=jnp.float32) + b_ref[...]
    y = jnp.where(rows < N_NODES, y, 0.0)
    y_ref[...] = y

    @pl.when(i == 0)
    def _():
        s_ref[...] = jnp.zeros_like(s_ref)
        q_ref[...] = jnp.zeros_like(q_ref)

    s_ref[...] += jnp.sum(y, axis=0, keepdims=True)
    q_ref[...] += jnp.sum(y * y, axis=0, keepdims=True)


def _kb(z1, a1, c1, w2, b2):
    e1 = z1.shape[1]
    e2 = w2.shape[1]
    return pl.pallas_call(
        _kb_body,
        grid=(_GRID,),
        in_specs=[
            pl.BlockSpec((_BLK, e1), lambda i: (i, 0)),
            pl.BlockSpec((1, e1), lambda i: (0, 0)),
            pl.BlockSpec((1, e1), lambda i: (0, 0)),
            pl.BlockSpec((e1, e2), lambda i: (0, 0)),
            pl.BlockSpec((1, e2), lambda i: (0, 0)),
        ],
        out_specs=[
            pl.BlockSpec((_BLK, e2), lambda i: (i, 0)),
            pl.BlockSpec((1, e2), lambda i: (0, 0)),
            pl.BlockSpec((1, e2), lambda i: (0, 0)),
        ],
        out_shape=[
            jax.ShapeDtypeStruct((NPAD, e2), jnp.float32),
            jax.ShapeDtypeStruct((1, e2), jnp.float32),
            jax.ShapeDtypeStruct((1, e2), jnp.float32),
        ],
    )(z1, a1, c1, w2, b2[None])


def _kc_body(y_ref, a_ref, c_ref, h_ref, *, relu):
    h = y_ref[...] * a_ref[...] + c_ref[...]
    if relu:
        h = jnp.maximum(h, 0.0)
    i = pl.program_id(0)
    rows = lax.broadcasted_iota(jnp.int32, (_BLK, 1), 0) + i * _BLK
    h_ref[...] = jnp.where(rows < N_NODES, h, 0.0)


def _kc(y, a2, c2, relu):
    return pl.pallas_call(
        functools.partial(_kc_body, relu=relu),
        grid=(_GRID,),
        in_specs=[
            pl.BlockSpec((_BLK, EMB), lambda i: (i, 0)),
            pl.BlockSpec((1, EMB), lambda i: (0, 0)),
            pl.BlockSpec((1, EMB), lambda i: (0, 0)),
        ],
        out_specs=pl.BlockSpec((_BLK, EMB), lambda i: (i, 0)),
        out_shape=jax.ShapeDtypeStruct((NPAD, EMB), jnp.float32),
    )(y, a2, c2)


def _fin_body(pool_ref, cnt_ref, g_ref):
    p = pool_ref[0, :NB, :] + pool_ref[1, :NB, :]
    cnt = jnp.sum(cnt_ref[...], axis=0)
    cnt = jnp.sum(cnt, axis=1, keepdims=True)
    g_ref[...] = p / (cnt[:NB] + 1e-9)


def _fin(pool, cnt):
    return pl.pallas_call(
        _fin_body,
        out_shape=jax.ShapeDtypeStruct((NB, EMB), jnp.float32),
    )(pool, cnt)


def kernel(x, edge_index, edge_attr, batch, atom_tables, bond_tables,
           W1, b1, bn1_g, bn1_b, W2, b2, eps_p, bn_g, bn_b):
    x = x.astype(jnp.int32)
    edge_index = edge_index.astype(jnp.int32)
    edge_attr = edge_attr.astype(jnp.int32)
    batch = batch.astype(jnp.int32)

    xoff = x + (jnp.arange(9, dtype=jnp.int32) * 120)[None, :]
    xp = jnp.pad(xoff, ((0, NPAD - N_NODES), (0, 0)))
    xt = xp.reshape(NROWS, 128, 9).transpose(0, 2, 1)
    atab = atom_tables.reshape(540, 128)
    src = edge_index[0]
    dst = edge_index[1]
    cidx = edge_attr[:, 0] + 5 * edge_attr[:, 1] + 25 * edge_attr[:, 2]
    pe = EPAD - N_EDGES
    src4 = jnp.pad(src, (0, pe)).reshape(NSC, 4, 64)
    dst4 = jnp.pad(dst, (0, pe), constant_values=2 ** 20).reshape(NSC, 4, 64)
    cidx4 = jnp.pad(cidx, (0, pe)).reshape(NSC, 4, 64)
    pack3 = jnp.stack([src4, dst4, cidx4], axis=2).reshape(NSC, 12, 64)
    bt = bond_tables
    ctab = (bt[:, 2][:, :, None, None, :] + bt[:, 1][:, None, :, None, :]
            + bt[:, 0][:, None, None, :, :])[:, :5, :5, :5, :]
    ctab = ctab.reshape(NLAYERS, 125, EMB)
    ctab = jnp.pad(ctab, ((0, 0), (0, 3), (0, 0))).reshape(NLAYERS, 64, 128)
    batchp = jnp.pad(batch, (0, NPAD - N_NODES),
                     constant_values=NB).reshape(NROWS, 128)

    h = _atom_call(xt, atab)
    for l in range(NLAYERS):
        agg = _edge_call(h, pack3, ctab[l])
        z1, s1, q1 = _ka(h, agg, jnp.reshape(eps_p[l], (1,)), W1[l], b1[l])
        a1, c1 = _bnfold(s1, q1, bn1_g[l], bn1_b[l])
        y, s2, q2 = _kb(z1, a1, c1, W2[l], b2[l])
        a2, c2 = _bnfold(s2, q2, bn_g[l], bn_b[l])
        h = _kc(y, a2, c2, relu=(l != NLAYERS - 1))

    pool, cnt = _pool_call(h, batchp)
    graph_repr = _fin(pool, cnt)
    node_repr = h[:N_NODES]
    return (node_repr, graph_repr)


# speedup vs baseline: 1.0364x; 1.0364x over previous
import functools

import jax
import jax.numpy as jnp
from jax import lax
from jax.experimental import pallas as pl
from jax.experimental.pallas import tpu as pltpu
from jax.experimental.pallas import tpu_sc as plsc

N_NODES = 50000
N_EDGES = 800000
EMB = 64
NLAYERS = 4
NB = 128

NC, NS = 2, 16
NPAD = 57344
NROWS = NPAD // 128
EPAD = 802816
EROWS = EPAD // 128
ER_PER_TILE = EROWS // NS
ECH = ER_PER_TILE // 4
HALF = N_NODES // 2
AGG_ROWS = 25088
SLAB = AGG_ROWS // NS
TRASH = HALF

_mesh = plsc.VectorSubcoreMesh(
    core_axis_name="c", subcore_axis_name="s", num_cores=NC, num_subcores=NS)
_SC_PARAMS = pltpu.CompilerParams(needs_layout_passes=False,
                                  use_tc_tiling_on_sc=False)


def _atom_body(xt_hbm, atab_hbm, h_hbm, atv, xv0, xv1, hb0, hb1,
               xsem0, xsem1, hsem0, hsem1):
    c = lax.axis_index("c")
    s = lax.axis_index("s")
    wid = s * NC + c
    pltpu.sync_copy(atab_hbm, atv)
    iota = lax.broadcasted_iota(jnp.int32, (16,), 0)
    xv = (xv0, xv1)
    hb = (hb0, hb1)
    xsem = (xsem0, xsem1)
    r0 = wid * 14

    def _compute(xvq, hbq):
        for g in range(8):
            xf = [xvq[f, pl.ds(g * 16, 16)] for f in range(9)]
            xrow = [x >> 1 for x in xf]
            xcol = [(x & 1) << 6 for x in xf]
            rows = g * 16 + iota

            @plsc.parallel_loop(0, EMB, unroll=4)
            def _col(col):
                colv = jnp.full((16,), col, jnp.int32)
                acc = plsc.load_gather(atv, [xrow[0], xcol[0] + colv])
                for f in range(1, 9):
                    acc = acc + plsc.load_gather(atv, [xrow[f], xcol[f] + colv])
                plsc.store_scatter(hbq, [rows, colv], acc)

    @pl.loop(0, 14, step=2)
    def _pair(g):
        dxa = pltpu.async_copy(xt_hbm.at[r0 + g], xv0, xsem0)
        dxb = pltpu.async_copy(xt_hbm.at[r0 + g + 1], xv1, xsem1)
        dxa.wait()
        _compute(xv0, hb0)
        dha = pltpu.async_copy(hb0, h_hbm.at[pl.ds((r0 + g) * 128, 128)],
                               hsem0)
        dxb.wait()
        _compute(xv1, hb1)
        dhb = pltpu.async_copy(hb1, h_hbm.at[pl.ds((r0 + g + 1) * 128, 128)],
                               hsem1)
        dha.wait()
        dhb.wait()


_atom_call = functools.partial(
    pl.kernel,
    out_type=jax.ShapeDtypeStruct((NPAD, EMB), jnp.float32),
    mesh=_mesh,
    compiler_params=_SC_PARAMS,
    scratch_types=[
        pltpu.VMEM((540, 128), jnp.float32),
        pltpu.VMEM((9, 128), jnp.int32),
        pltpu.VMEM((9, 128), jnp.int32),
        pltpu.VMEM((128, EMB), jnp.float32),
        pltpu.VMEM((128, EMB), jnp.float32),
        pltpu.SemaphoreType.DMA,
        pltpu.SemaphoreType.DMA,
        pltpu.SemaphoreType.DMA,
        pltpu.SemaphoreType.DMA,
    ],
)(_atom_body)


NSC = 3136
SC_PER_TILE = NSC // NS


def _edge_body(h_hbm, pk_hbm, ctab_hbm, agg_hbm,
               ctv, idxv0, idxv1, grows0, grows1, msgb0, msgb1, aggs,
               gsem0, gsem1, ssem0, ssem1):
    c = lax.axis_index("c")
    s = lax.axis_index("s")
    base = c * HALF
    pltpu.sync_copy(ctab_hbm, ctv)
    zvec = jnp.zeros((16,), jnp.float32)

    @pl.loop(0, 64)
    def _zg(i):
        for q in range(4):
            msgb0[i, pl.ds(q * 16, 16)] = zvec

    for j in range(24):
        pltpu.sync_copy(msgb0, aggs.at[pl.ds(s * SLAB + j * 64, 64)])
    pltpu.sync_copy(msgb0.at[pl.ds(0, 32)],
                    aggs.at[pl.ds(s * SLAB + 1536, 32)])
    plsc.subcore_barrier()

    iota = lax.broadcasted_iota(jnp.int32, (16,), 0)
    idxv = (idxv0, idxv1)
    grows = (grows0, grows1)
    msgb = (msgb0, msgb1)
    gsem = (gsem0, gsem1)
    ssem = (ssem0, ssem1)
    sc0 = s * SC_PER_TILE

    def _compute(G, M, ip, b):
        for g4 in range(4):
            sl = pl.ds(g4 * 16, 16)
            d = ip[3 * b + 1, sl]
            own = (d >= base) & (d < base + HALF)
            ip[3 * b + 1, sl] = jnp.where(own, d - base, TRASH + (d & 63))
            cidx16 = ip[3 * b + 2, sl]
            crow = cidx16 >> 1
            ccol = (cidx16 & 1) << 6
            rows = g4 * 16 + iota

            @plsc.parallel_loop(0, EMB, unroll=4)
            def _col(col):
                colv = jnp.full((16,), col, jnp.int32)
                hv = plsc.load_gather(G, [rows, colv])
                ev = plsc.load_gather(ctv, [crow, ccol + colv])
                plsc.store_scatter(M, [rows, colv],
                                   jnp.maximum(hv + ev, 0.0))

    @pl.loop(0, SC_PER_TILE, step=2)
    def _super(g):
        sc = sc0 + g
        pltpu.sync_copy(pk_hbm.at[sc], idxv0)
        pltpu.sync_copy(pk_hbm.at[sc + 1], idxv1)
        dg = [None] * 8
        ds_ = [None] * 8
        dg[0] = pltpu.async_copy(h_hbm.at[idxv0.at[0]], grows0, gsem0)
        for t in range(8):
            q = t & 1
            b = t % 4
            ip = idxv[t // 4]
            if t < 7:
                nip = idxv[(t + 1) // 4]
                nb = (t + 1) % 4
                dg[t + 1] = pltpu.async_copy(h_hbm.at[nip.at[3 * nb]],
                                             grows[1 - q], gsem[1 - q])
            dg[t].wait()
            if t >= 2:
                ds_[t - 2].wait()
            _compute(grows[q], msgb[q], ip, b)
            ds_[t] = pltpu.async_copy(msgb[q], aggs.at[ip.at[3 * b + 1]],
                                      ssem[q], add=True)
        ds_[6].wait()
        ds_[7].wait()

    plsc.subcore_barrier()

    @pl.when(s < NS - 1)
    def _full_slab():
        pltpu.sync_copy(aggs.at[pl.ds(s * SLAB, SLAB)],
                        agg_hbm.at[pl.ds(base + s * SLAB, SLAB)])

    @pl.when(s == NS - 1)
    def _last_slab():
        pltpu.sync_copy(aggs.at[pl.ds((NS - 1) * SLAB, HALF - (NS - 1) * SLAB)],
                        agg_hbm.at[pl.ds(base + (NS - 1) * SLAB,
                                         HALF - (NS - 1) * SLAB)])


_edge_call = functools.partial(
    pl.kernel,
    out_type=jax.ShapeDtypeStruct((NPAD, EMB), jnp.float32),
    mesh=_mesh,
    compiler_params=_SC_PARAMS,
    scratch_types=[
        pltpu.VMEM((64, 128), jnp.float32),
        pltpu.VMEM((12, 64), jnp.int32),
        pltpu.VMEM((12, 64), jnp.int32),
        pltpu.VMEM((64, EMB), jnp.float32),
        pltpu.VMEM((64, EMB), jnp.float32),
        pltpu.VMEM((64, EMB), jnp.float32),
        pltpu.VMEM((64, EMB), jnp.float32),
        pltpu.VMEM_SHARED((AGG_ROWS, EMB), jnp.float32),
        pltpu.SemaphoreType.DMA,
        pltpu.SemaphoreType.DMA,
        pltpu.SemaphoreType.DMA,
        pltpu.SemaphoreType.DMA,
    ],
)(_edge_body)


def _pool_body(nr_hbm, b_hbm, pool_hbm, cnt_hbm, vbuf, bv, cntv, zb9, pools):
    c = lax.axis_index("c")
    s = lax.axis_index("s")
    wid = s * NC + c
    iota = lax.broadcasted_iota(jnp.int32, (16,), 0)
    ones = jnp.ones((16,), jnp.float32)
    zvec = jnp.zeros((16,), jnp.float32)

    @pl.loop(0, 144)
    def _zc(i):
        cntv[i, pl.ds(0, 16)] = zvec

    @pl.loop(0, 9)
    def _z9(i):
        for q in range(4):
            zb9[i, pl.ds(q * 16, 16)] = zvec

    pltpu.sync_copy(zb9, pools.at[pl.ds(s * 9, 9)])
    plsc.subcore_barrier()

    @pl.loop(0, 14)
    def _chunk(i):
        r = wid * 14 + i
        pltpu.sync_copy(nr_hbm.at[pl.ds(r * 128, 128)], vbuf)
        pltpu.sync_copy(b_hbm.at[r], bv)
        pltpu.sync_copy(vbuf, pools.at[bv], add=True)
        for g in range(8):
            b16 = bv[pl.ds(g * 16, 16)]
            plsc.addupdate_scatter(cntv, [b16, iota], ones)

    plsc.subcore_barrier()

    @pl.when(s == 0)
    def _out_pool():
        pltpu.sync_copy(pools, pool_hbm.at[c])

    pltpu.sync_copy(cntv, cnt_hbm.at[wid])


_pool_call = functools.partial(
    pl.kernel,
    out_type=(jax.ShapeDtypeStruct((NC, 144, EMB), jnp.float32),
              jax.ShapeDtypeStruct((NC * NS, 144, 16), jnp.float32)),
    mesh=_mesh,
    compiler_params=_SC_PARAMS,
    scratch_types=[
        pltpu.VMEM((128, EMB), jnp.float32),
        pltpu.VMEM((128,), jnp.int32),
        pltpu.VMEM((144, 16), jnp.float32),
        pltpu.VMEM((9, EMB), jnp.float32),
        pltpu.VMEM_SHARED((144, EMB), jnp.float32),
    ],
)(_pool_body)


_BLK = 1024
_GRID = NPAD // _BLK


def _ka_body(eps_ref, h_ref, agg_ref, w_ref, b_ref, z1_ref, s_ref, q_ref):
    i = pl.program_id(0)
    e = eps_ref[0]
    z = (1.0 + e) * h_ref[...] + agg_ref[...]
    z1 = lax.dot_general(z, w_ref[...], (((1,), (0,)), ((), ())),
                         preferred_element_type=jnp.float32) + b_ref[...]
    rows = lax.broadcasted_iota(jnp.int32, (_BLK, 1), 0) + i * _BLK
    z1 = jnp.where(rows < N_NODES, z1, 0.0)
    z1_ref[...] = z1

    @pl.when(i == 0)
    def _():
        s_ref[...] = jnp.zeros_like(s_ref)
        q_ref[...] = jnp.zeros_like(q_ref)

    s_ref[...] += jnp.sum(z1, axis=0, keepdims=True)
    q_ref[...] += jnp.sum(z1 * z1, axis=0, keepdims=True)


def _ka(h, agg, eps, w1, b1):
    e2 = w1.shape[1]
    return pl.pallas_call(
        _ka_body,
        grid=(_GRID,),
        in_specs=[
            pl.BlockSpec(memory_space=pltpu.SMEM),
            pl.BlockSpec((_BLK, EMB), lambda i: (i, 0)),
            pl.BlockSpec((_BLK, EMB), lambda i: (i, 0)),
            pl.BlockSpec((EMB, e2), lambda i: (0, 0)),
            pl.BlockSpec((1, e2), lambda i: (0, 0)),
        ],
        out_specs=[
            pl.BlockSpec((_BLK, e2), lambda i: (i, 0)),
            pl.BlockSpec((1, e2), lambda i: (0, 0)),
            pl.BlockSpec((1, e2), lambda i: (0, 0)),
        ],
        out_shape=[
            jax.ShapeDtypeStruct((NPAD, e2), jnp.float32),
            jax.ShapeDtypeStruct((1, e2), jnp.float32),
            jax.ShapeDtypeStruct((1, e2), jnp.float32),
        ],
    )(eps, h, agg, w1, b1[None])


def _bnfold_body(s_ref, q_ref, g_ref, beta_ref, a_ref, c_ref):
    inv_n = 1.0 / N_NODES
    m = s_ref[...] * inv_n
    v = q_ref[...] * inv_n - m * m
    a = g_ref[...] * lax.rsqrt(v + 1e-5)
    a_ref[...] = a
    c_ref[...] = beta_ref[...] - m * a


def _bnfold(s, q, g, beta):
    e2 = s.shape[1]
    return pl.pallas_call(
        _bnfold_body,
        out_shape=[
            jax.ShapeDtypeStruct((1, e2), jnp.float32),
            jax.ShapeDtypeStruct((1, e2), jnp.float32),
        ],
    )(s, q, g[None], beta[None])


def _kb_body(z1_ref, a_ref, c_ref, w_ref, b_ref, y_ref, s_ref, q_ref):
    i = pl.program_id(0)
    r = jnp.maximum(z1_ref[...] * a_ref[...] + c_ref[...], 0.0)
    rows = lax.broadcasted_iota(jnp.int32, (_BLK, 1), 0) + i * _BLK
    r = jnp.where(rows < N_NODES, r, 0.0)
    y = lax.dot_general(r, w_ref[...], (((1,), (0,)), ((), ())),
                        preferred_element_type=jnp.float32) + b_ref[...]
    y = jnp.where(rows < N_NODES, y, 0.0)
    y_ref[...] = y

    @pl.when(i == 0)
    def _():
        s_ref[...] = jnp.zeros_like(s_ref)
        q_ref[...] = jnp.zeros_like(q_ref)

    s_ref[...] += jnp.sum(y, axis=0, keepdims=True)
    q_ref[...] += jnp.sum(y * y, axis=0, keepdims=True)


def _kb(z1, a1, c1, w2, b2):
    e1 = z1.shape[1]
    e2 = w2.shape[1]
    return pl.pallas_call(
        _kb_body,
        grid=(_GRID,),
        in_specs=[
            pl.BlockSpec((_BLK, e1), lambda i: (i, 0)),
            pl.BlockSpec((1, e1), lambda i: (0, 0)),
            pl.BlockSpec((1, e1), lambda i: (0, 0)),
            pl.BlockSpec((e1, e2), lambda i: (0, 0)),
            pl.BlockSpec((1, e2), lambda i: (0, 0)),
        ],
        out_specs=[
            pl.BlockSpec((_BLK, e2), lambda i: (i, 0)),
            pl.BlockSpec((1, e2), lambda i: (0, 0)),
            pl.BlockSpec((1, e2), lambda i: (0, 0)),
        ],
        out_shape=[
            jax.ShapeDtypeStruct((NPAD, e2), jnp.float32),
            jax.ShapeDtypeStruct((1, e2), jnp.float32),
            jax.ShapeDtypeStruct((1, e2), jnp.float32),
        ],
    )(z1, a1, c1, w2, b2[None])


def _kc_body(y_ref, a_ref, c_ref, h_ref, *, relu):
    h = y_ref[...] * a_ref[...] + c_ref[...]
    if relu:
        h = jnp.maximum(h, 0.0)
    i = pl.program_id(0)
    rows = lax.broadcasted_iota(jnp.int32, (_BLK, 1), 0) + i * _BLK
    h_ref[...] = jnp.where(rows < N_NODES, h, 0.0)


def _kc(y, a2, c2, relu):
    return pl.pallas_call(
        functools.partial(_kc_body, relu=relu),
        grid=(_GRID,),
        in_specs=[
            pl.BlockSpec((_BLK, EMB), lambda i: (i, 0)),
            pl.BlockSpec((1, EMB), lambda i: (0, 0)),
            pl.BlockSpec((1, EMB), lambda i: (0, 0)),
        ],
        out_specs=pl.BlockSpec((_BLK, EMB), lambda i: (i, 0)),
        out_shape=jax.ShapeDtypeStruct((NPAD, EMB), jnp.float32),
    )(y, a2, c2)


def _fin_body(pool_ref, cnt_ref, g_ref):
    p = pool_ref[0, :NB, :] + pool_ref[1, :NB, :]
    cnt = jnp.sum(cnt_ref[...], axis=0)
    cnt = jnp.sum(cnt, axis=1, keepdims=True)
    g_ref[...] = p / (cnt[:NB] + 1e-9)


def _fin(pool, cnt):
    return pl.pallas_call(
        _fin_body,
        out_shape=jax.ShapeDtypeStruct((NB, EMB), jnp.float32),
    )(pool, cnt)


def kernel(x, edge_index, edge_attr, batch, atom_tables, bond_tables,
           W1, b1, bn1_g, bn1_b, W2, b2, eps_p, bn_g, bn_b):
    x = x.astype(jnp.int32)
    edge_index = edge_index.astype(jnp.int32)
    edge_attr = edge_attr.astype(jnp.int32)
    batch = batch.astype(jnp.int32)

    xoff = x + (jnp.arange(9, dtype=jnp.int32) * 120)[None, :]
    xp = jnp.pad(xoff, ((0, NPAD - N_NODES), (0, 0)))
    xt = xp.reshape(NROWS, 128, 9).transpose(0, 2, 1)
    atab = atom_tables.reshape(540, 128)
    src = edge_index[0]
    dst = edge_index[1]
    cidx = edge_attr[:, 0] + 5 * edge_attr[:, 1] + 25 * edge_attr[:, 2]
    pe = EPAD - N_EDGES
    src4 = jnp.pad(src, (0, pe)).reshape(NSC, 4, 64)
    dst4 = jnp.pad(dst, (0, pe), constant_values=2 ** 20).reshape(NSC, 4, 64)
    cidx4 = jnp.pad(cidx, (0, pe)).reshape(NSC, 4, 64)
    pack3 = jnp.stack([src4, dst4, cidx4], axis=2).reshape(NSC, 12, 64)
    bt = bond_tables
    ctab = (bt[:, 2][:, :, None, None, :] + bt[:, 1][:, None, :, None, :]
            + bt[:, 0][:, None, None, :, :])[:, :5, :5, :5, :]
    ctab = ctab.reshape(NLAYERS, 125, EMB)
    ctab = jnp.pad(ctab, ((0, 0), (0, 3), (0, 0))).reshape(NLAYERS, 64, 128)
    batchp = jnp.pad(batch, (0, NPAD - N_NODES),
                     constant_values=NB).reshape(NROWS, 128)

    h = _atom_call(xt, atab)
    for l in range(NLAYERS):
        agg = _edge_call(h, pack3, ctab[l])
        z1, s1, q1 = _ka(h, agg, jnp.reshape(eps_p[l], (1,)), W1[l], b1[l])
        a1, c1 = _bnfold(s1, q1, bn1_g[l], bn1_b[l])
        y, s2, q2 = _kb(z1, a1, c1, W2[l], b2[l])
        a2, c2 = _bnfold(s2, q2, bn_g[l], bn_b[l])
        h = _kc(y, a2, c2, relu=(l != NLAYERS - 1))

    pool, cnt = _pool_call(h, batchp)
    graph_repr = _fin(pool, cnt)
    node_repr = h[:N_NODES]
    return (node_repr, graph_repr)


# speedup vs baseline: 1.0572x; 1.0201x over previous
import functools

import jax
import jax.numpy as jnp
from jax import lax
from jax.experimental import pallas as pl
from jax.experimental.pallas import tpu as pltpu
from jax.experimental.pallas import tpu_sc as plsc

N_NODES = 50000
N_EDGES = 800000
EMB = 64
NLAYERS = 4
NB = 128

NC, NS = 2, 16
NPAD = 57344
NROWS = NPAD // 128
EPAD = 802816
EROWS = EPAD // 128
ER_PER_TILE = EROWS // NS
ECH = ER_PER_TILE // 4
HALF = N_NODES // 2
AGG_ROWS = 25088
SLAB = AGG_ROWS // NS
TRASH = HALF

_mesh = plsc.VectorSubcoreMesh(
    core_axis_name="c", subcore_axis_name="s", num_cores=NC, num_subcores=NS)
_SC_PARAMS = pltpu.CompilerParams(needs_layout_passes=False,
                                  use_tc_tiling_on_sc=False)


def _atom_body(xt_hbm, atab_hbm, h_hbm, atv, xv0, xv1, hb0, hb1,
               xsem0, xsem1, hsem0, hsem1):
    c = lax.axis_index("c")
    s = lax.axis_index("s")
    wid = s * NC + c
    pltpu.sync_copy(atab_hbm, atv)
    iota = lax.broadcasted_iota(jnp.int32, (16,), 0)
    xv = (xv0, xv1)
    hb = (hb0, hb1)
    xsem = (xsem0, xsem1)
    r0 = wid * 14

    def _compute(xvq, hbq):
        for g in range(8):
            xf = [xvq[f, pl.ds(g * 16, 16)] for f in range(9)]
            xrow = [x >> 1 for x in xf]
            xcol = [(x & 1) << 6 for x in xf]
            rows = g * 16 + iota

            @plsc.parallel_loop(0, EMB, unroll=4)
            def _col(col):
                colv = jnp.full((16,), col, jnp.int32)
                acc = plsc.load_gather(atv, [xrow[0], xcol[0] + colv])
                for f in range(1, 9):
                    acc = acc + plsc.load_gather(atv, [xrow[f], xcol[f] + colv])
                plsc.store_scatter(hbq, [rows, colv], acc)

    @pl.loop(0, 14, step=2)
    def _pair(g):
        dxa = pltpu.async_copy(xt_hbm.at[r0 + g], xv0, xsem0)
        dxb = pltpu.async_copy(xt_hbm.at[r0 + g + 1], xv1, xsem1)
        dxa.wait()
        _compute(xv0, hb0)
        dha = pltpu.async_copy(hb0, h_hbm.at[pl.ds((r0 + g) * 128, 128)],
                               hsem0)
        dxb.wait()
        _compute(xv1, hb1)
        dhb = pltpu.async_copy(hb1, h_hbm.at[pl.ds((r0 + g + 1) * 128, 128)],
                               hsem1)
        dha.wait()
        dhb.wait()


_atom_call = functools.partial(
    pl.kernel,
    out_type=jax.ShapeDtypeStruct((NPAD, EMB), jnp.float32),
    mesh=_mesh,
    compiler_params=_SC_PARAMS,
    scratch_types=[
        pltpu.VMEM((540, 128), jnp.float32),
        pltpu.VMEM((9, 128), jnp.int32),
        pltpu.VMEM((9, 128), jnp.int32),
        pltpu.VMEM((128, EMB), jnp.float32),
        pltpu.VMEM((128, EMB), jnp.float32),
        pltpu.SemaphoreType.DMA,
        pltpu.SemaphoreType.DMA,
        pltpu.SemaphoreType.DMA,
        pltpu.SemaphoreType.DMA,
    ],
)(_atom_body)


NSC = 3136
SC_PER_TILE = NSC // NS


def _edge_body(h_hbm, pk_hbm, ctab_hbm, agg_hbm,
               ctv, idxv0, idxv1, grows0, grows1, msgb0, msgb1, aggs,
               gsem0, gsem1, ssem0, ssem1):
    c = lax.axis_index("c")
    s = lax.axis_index("s")
    base = c * HALF
    pltpu.sync_copy(ctab_hbm, ctv)
    zvec = jnp.zeros((16,), jnp.float32)

    @pl.loop(0, 64)
    def _zg(i):
        for q in range(4):
            msgb0[i, pl.ds(q * 16, 16)] = zvec

    for j in range(24):
        pltpu.sync_copy(msgb0, aggs.at[pl.ds(s * SLAB + j * 64, 64)])
    pltpu.sync_copy(msgb0.at[pl.ds(0, 32)],
                    aggs.at[pl.ds(s * SLAB + 1536, 32)])
    plsc.subcore_barrier()

    iota = lax.broadcasted_iota(jnp.int32, (16,), 0)
    idxv = (idxv0, idxv1)
    grows = (grows0, grows1)
    msgb = (msgb0, msgb1)
    gsem = (gsem0, gsem1)
    ssem = (ssem0, ssem1)
    sc0 = s * SC_PER_TILE

    def _compute(G, M, ip, b):
        for g4 in range(4):
            sl = pl.ds(g4 * 16, 16)
            d = ip[3 * b + 1, sl]
            own = (d >= base) & (d < base + HALF)
            ip[3 * b + 1, sl] = jnp.where(own, d - base, TRASH + (d & 63))
            cidx16 = ip[3 * b + 2, sl]
            crow = cidx16 >> 1
            ccol = (cidx16 & 1) << 6
            rows = g4 * 16 + iota

            @plsc.parallel_loop(0, EMB, unroll=8)
            def _col(col):
                colv = jnp.full((16,), col, jnp.int32)
                hv = plsc.load_gather(G, [rows, colv])
                ev = plsc.load_gather(ctv, [crow, ccol + colv])
                plsc.store_scatter(M, [rows, colv],
                                   jnp.maximum(hv + ev, 0.0))

    @pl.loop(0, SC_PER_TILE, step=2)
    def _super(g):
        sc = sc0 + g
        pltpu.sync_copy(pk_hbm.at[sc], idxv0)
        pltpu.sync_copy(pk_hbm.at[sc + 1], idxv1)
        dg = [None] * 8
        ds_ = [None] * 8
        dg[0] = pltpu.async_copy(h_hbm.at[idxv0.at[0]], grows0, gsem0)
        for t in range(8):
            q = t & 1
            b = t % 4
            ip = idxv[t // 4]
            if t < 7:
                nip = idxv[(t + 1) // 4]
                nb = (t + 1) % 4
                dg[t + 1] = pltpu.async_copy(h_hbm.at[nip.at[3 * nb]],
                                             grows[1 - q], gsem[1 - q])
            dg[t].wait()
            if t >= 2:
                ds_[t - 2].wait()
            _compute(grows[q], msgb[q], ip, b)
            ds_[t] = pltpu.async_copy(msgb[q], aggs.at[ip.at[3 * b + 1]],
                                      ssem[q], add=True)
        ds_[6].wait()
        ds_[7].wait()

    plsc.subcore_barrier()

    @pl.when(s < NS - 1)
    def _full_slab():
        pltpu.sync_copy(aggs.at[pl.ds(s * SLAB, SLAB)],
                        agg_hbm.at[pl.ds(base + s * SLAB, SLAB)])

    @pl.when(s == NS - 1)
    def _last_slab():
        pltpu.sync_copy(aggs.at[pl.ds((NS - 1) * SLAB, HALF - (NS - 1) * SLAB)],
                        agg_hbm.at[pl.ds(base + (NS - 1) * SLAB,
                                         HALF - (NS - 1) * SLAB)])


_edge_call = functools.partial(
    pl.kernel,
    out_type=jax.ShapeDtypeStruct((NPAD, EMB), jnp.float32),
    mesh=_mesh,
    compiler_params=_SC_PARAMS,
    scratch_types=[
        pltpu.VMEM((64, 128), jnp.float32),
        pltpu.VMEM((12, 64), jnp.int32),
        pltpu.VMEM((12, 64), jnp.int32),
        pltpu.VMEM((64, EMB), jnp.float32),
        pltpu.VMEM((64, EMB), jnp.float32),
        pltpu.VMEM((64, EMB), jnp.float32),
        pltpu.VMEM((64, EMB), jnp.float32),
        pltpu.VMEM_SHARED((AGG_ROWS, EMB), jnp.float32),
        pltpu.SemaphoreType.DMA,
        pltpu.SemaphoreType.DMA,
        pltpu.SemaphoreType.DMA,
        pltpu.SemaphoreType.DMA,
    ],
)(_edge_body)


def _pool_body(nr_hbm, b_hbm, pool_hbm, cnt_hbm, vbuf, bv, cntv, zb9, pools):
    c = lax.axis_index("c")
    s = lax.axis_index("s")
    wid = s * NC + c
    iota = lax.broadcasted_iota(jnp.int32, (16,), 0)
    ones = jnp.ones((16,), jnp.float32)
    zvec = jnp.zeros((16,), jnp.float32)

    @pl.loop(0, 144)
    def _zc(i):
        cntv[i, pl.ds(0, 16)] = zvec

    @pl.loop(0, 9)
    def _z9(i):
        for q in range(4):
            zb9[i, pl.ds(q * 16, 16)] = zvec

    pltpu.sync_copy(zb9, pools.at[pl.ds(s * 9, 9)])
    plsc.subcore_barrier()

    @pl.loop(0, 14)
    def _chunk(i):
        r = wid * 14 + i
        pltpu.sync_copy(nr_hbm.at[pl.ds(r * 128, 128)], vbuf)
        pltpu.sync_copy(b_hbm.at[r], bv)
        pltpu.sync_copy(vbuf, pools.at[bv], add=True)
        for g in range(8):
            b16 = bv[pl.ds(g * 16, 16)]
            plsc.addupdate_scatter(cntv, [b16, iota], ones)

    plsc.subcore_barrier()

    @pl.when(s == 0)
    def _out_pool():
        pltpu.sync_copy(pools, pool_hbm.at[c])

    pltpu.sync_copy(cntv, cnt_hbm.at[wid])


_pool_call = functools.partial(
    pl.kernel,
    out_type=(jax.ShapeDtypeStruct((NC, 144, EMB), jnp.float32),
              jax.ShapeDtypeStruct((NC * NS, 144, 16), jnp.float32)),
    mesh=_mesh,
    compiler_params=_SC_PARAMS,
    scratch_types=[
        pltpu.VMEM((128, EMB), jnp.float32),
        pltpu.VMEM((128,), jnp.int32),
        pltpu.VMEM((144, 16), jnp.float32),
        pltpu.VMEM((9, EMB), jnp.float32),
        pltpu.VMEM_SHARED((144, EMB), jnp.float32),
    ],
)(_pool_body)


_BLK = 1024
_GRID = NPAD // _BLK


def _ka_body(eps_ref, h_ref, agg_ref, w_ref, b_ref, z1_ref, s_ref, q_ref):
    i = pl.program_id(0)
    e = eps_ref[0]
    z = (1.0 + e) * h_ref[...] + agg_ref[...]
    z1 = lax.dot_general(z, w_ref[...], (((1,), (0,)), ((), ())),
                         preferred_element_type=jnp.float32) + b_ref[...]
    rows = lax.broadcasted_iota(jnp.int32, (_BLK, 1), 0) + i * _BLK
    z1 = jnp.where(rows < N_NODES, z1, 0.0)
    z1_ref[...] = z1

    @pl.when(i == 0)
    def _():
        s_ref[...] = jnp.zeros_like(s_ref)
        q_ref[...] = jnp.zeros_like(q_ref)

    s_ref[...] += jnp.sum(z1, axis=0, keepdims=True)
    q_ref[...] += jnp.sum(z1 * z1, axis=0, keepdims=True)


def _ka(h, agg, eps, w1, b1):
    e2 = w1.shape[1]
    return pl.pallas_call(
        _ka_body,
        grid=(_GRID,),
        in_specs=[
            pl.BlockSpec(memory_space=pltpu.SMEM),
            pl.BlockSpec((_BLK, EMB), lambda i: (i, 0)),
            pl.BlockSpec((_BLK, EMB), lambda i: (i, 0)),
            pl.BlockSpec((EMB, e2), lambda i: (0, 0)),
            pl.BlockSpec((1, e2), lambda i: (0, 0)),
        ],
        out_specs=[
            pl.BlockSpec((_BLK, e2), lambda i: (i, 0)),
            pl.BlockSpec((1, e2), lambda i: (0, 0)),
            pl.BlockSpec((1, e2), lambda i: (0, 0)),
        ],
        out_shape=[
            jax.ShapeDtypeStruct((NPAD, e2), jnp.float32),
            jax.ShapeDtypeStruct((1, e2), jnp.float32),
            jax.ShapeDtypeStruct((1, e2), jnp.float32),
        ],
    )(eps, h, agg, w1, b1[None])


def _bnfold_body(s_ref, q_ref, g_ref, beta_ref, a_ref, c_ref):
    inv_n = 1.0 / N_NODES
    m = s_ref[...] * inv_n
    v = q_ref[...] * inv_n - m * m
    a = g_ref[...] * lax.rsqrt(v + 1e-5)
    a_ref[...] = a
    c_ref[...] = beta_ref[...] - m * a


def _bnfold(s, q, g, beta):
    e2 = s.shape[1]
    return pl.pallas_call(
        _bnfold_body,
        out_shape=[
            jax.ShapeDtypeStruct((1, e2), jnp.float32),
            jax.ShapeDtypeStruct((1, e2), jnp.float32),
        ],
    )(s, q, g[None], beta[None])


def _kb_body(z1_ref, a_ref, c_ref, w_ref, b_ref, y_ref, s_ref, q_ref):
    i = pl.program_id(0)
    r = jnp.maximum(z1_ref[...] * a_ref[...] + c_ref[...], 0.0)
    rows = lax.broadcasted_iota(jnp.int32, (_BLK, 1), 0) + i * _BLK
    r = jnp.where(rows < N_NODES, r, 0.0)
    y = lax.dot_general(r, w_ref[...], (((1,), (0,)), ((), ())),
                        preferred_element_type=jnp.float32) + b_ref[...]
    y = jnp.where(rows < N_NODES, y, 0.0)
    y_ref[...] = y

    @pl.when(i == 0)
    def _():
        s_ref[...] = jnp.zeros_like(s_ref)
        q_ref[...] = jnp.zeros_like(q_ref)

    s_ref[...] += jnp.sum(y, axis=0, keepdims=True)
    q_ref[...] += jnp.sum(y * y, axis=0, keepdims=True)


def _kb(z1, a1, c1, w2, b2):
    e1 = z1.shape[1]
    e2 = w2.shape[1]
    return pl.pallas_call(
        _kb_body,
        grid=(_GRID,),
        in_specs=[
            pl.BlockSpec((_BLK, e1), lambda i: (i, 0)),
            pl.BlockSpec((1, e1), lambda i: (0, 0)),
            pl.BlockSpec((1, e1), lambda i: (0, 0)),
            pl.BlockSpec((e1, e2), lambda i: (0, 0)),
            pl.BlockSpec((1, e2), lambda i: (0, 0)),
        ],
        out_specs=[
            pl.BlockSpec((_BLK, e2), lambda i: (i, 0)),
            pl.BlockSpec((1, e2), lambda i: (0, 0)),
            pl.BlockSpec((1, e2), lambda i: (0, 0)),
        ],
        out_shape=[
            jax.ShapeDtypeStruct((NPAD, e2), jnp.float32),
            jax.ShapeDtypeStruct((1, e2), jnp.float32),
            jax.ShapeDtypeStruct((1, e2), jnp.float32),
        ],
    )(z1, a1, c1, w2, b2[None])


def _kc_body(y_ref, a_ref, c_ref, h_ref, *, relu):
    h = y_ref[...] * a_ref[...] + c_ref[...]
    if relu:
        h = jnp.maximum(h, 0.0)
    i = pl.program_id(0)
    rows = lax.broadcasted_iota(jnp.int32, (_BLK, 1), 0) + i * _BLK
    h_ref[...] = jnp.where(rows < N_NODES, h, 0.0)


def _kc(y, a2, c2, relu):
    return pl.pallas_call(
        functools.partial(_kc_body, relu=relu),
        grid=(_GRID,),
        in_specs=[
            pl.BlockSpec((_BLK, EMB), lambda i: (i, 0)),
            pl.BlockSpec((1, EMB), lambda i: (0, 0)),
            pl.BlockSpec((1, EMB), lambda i: (0, 0)),
        ],
        out_specs=pl.BlockSpec((_BLK, EMB), lambda i: (i, 0)),
        out_shape=jax.ShapeDtypeStruct((NPAD, EMB), jnp.float32),
    )(y, a2, c2)


def _fin_body(pool_ref, cnt_ref, g_ref):
    p = pool_ref[0, :NB, :] + pool_ref[1, :NB, :]
    cnt = jnp.sum(cnt_ref[...], axis=0)
    cnt = jnp.sum(cnt, axis=1, keepdims=True)
    g_ref[...] = p / (cnt[:NB] + 1e-9)


def _fin(pool, cnt):
    return pl.pallas_call(
        _fin_body,
        out_shape=jax.ShapeDtypeStruct((NB, EMB), jnp.float32),
    )(pool, cnt)


def kernel(x, edge_index, edge_attr, batch, atom_tables, bond_tables,
           W1, b1, bn1_g, bn1_b, W2, b2, eps_p, bn_g, bn_b):
    x = x.astype(jnp.int32)
    edge_index = edge_index.astype(jnp.int32)
    edge_attr = edge_attr.astype(jnp.int32)
    batch = batch.astype(jnp.int32)

    xoff = x + (jnp.arange(9, dtype=jnp.int32) * 120)[None, :]
    xp = jnp.pad(xoff, ((0, NPAD - N_NODES), (0, 0)))
    xt = xp.reshape(NROWS, 128, 9).transpose(0, 2, 1)
    atab = atom_tables.reshape(540, 128)
    src = edge_index[0]
    dst = edge_index[1]
    cidx = edge_attr[:, 0] + 5 * edge_attr[:, 1] + 25 * edge_attr[:, 2]
    pe = EPAD - N_EDGES
    src4 = jnp.pad(src, (0, pe)).reshape(NSC, 4, 64)
    dst4 = jnp.pad(dst, (0, pe), constant_values=2 ** 20).reshape(NSC, 4, 64)
    cidx4 = jnp.pad(cidx, (0, pe)).reshape(NSC, 4, 64)
    pack3 = jnp.stack([src4, dst4, cidx4], axis=2).reshape(NSC, 12, 64)
    bt = bond_tables
    ctab = (bt[:, 2][:, :, None, None, :] + bt[:, 1][:, None, :, None, :]
            + bt[:, 0][:, None, None, :, :])[:, :5, :5, :5, :]
    ctab = ctab.reshape(NLAYERS, 125, EMB)
    ctab = jnp.pad(ctab, ((0, 0), (0, 3), (0, 0))).reshape(NLAYERS, 64, 128)
    batchp = jnp.pad(batch, (0, NPAD - N_NODES),
                     constant_values=NB).reshape(NROWS, 128)

    h = _atom_call(xt, atab)
    for l in range(NLAYERS):
        agg = _edge_call(h, pack3, ctab[l])
        z1, s1, q1 = _ka(h, agg, jnp.reshape(eps_p[l], (1,)), W1[l], b1[l])
        a1, c1 = _bnfold(s1, q1, bn1_g[l], bn1_b[l])
        y, s2, q2 = _kb(z1, a1, c1, W2[l], b2[l])
        a2, c2 = _bnfold(s2, q2, bn_g[l], bn_b[l])
        h = _kc(y, a2, c2, relu=(l != NLAYERS - 1))

    pool, cnt = _pool_call(h, batchp)
    graph_repr = _fin(pool, cnt)
    node_repr = h[:N_NODES]
    return (node_repr, graph_repr)
